# Initial kernel scaffold; baseline (speedup 1.0000x reference)
import jax
import jax.numpy as jnp
from jax.experimental import pallas as pl


def kernel(types1, types2, emb, W_iou, U_iou, b_iou, U_f_w, U_f_b, classify_w, classify_b):
    raise NotImplementedError("write your pallas kernel here")


# speedup vs baseline: 14.6682x; 14.6682x over previous
import functools

import jax
import jax.numpy as jnp
from jax import lax
from jax.experimental import pallas as pl
from jax.experimental.pallas import tpu as pltpu
from jax.experimental.pallas import tpu_sc as plsc

N = 50000
H = 128
K = 32
GS = 1057
NCHUNK = 48
CHUNK = 1024
GL = NCHUNK * CHUNK
PA = 1600
LEAF0 = 1563

SC_NC = 2
SC_NS = 16
NW = SC_NC * SC_NS
RPW = 2 * GL // NW
GCH = 128
GITER = RPW // GCH


def _gather_rows(emb, idx2d):
    mesh = plsc.VectorSubcoreMesh(core_axis_name="c", subcore_axis_name="s")

    @functools.partial(
        pl.kernel,
        mesh=mesh,
        out_type=jax.ShapeDtypeStruct((2 * GL, H), jnp.float32),
        scratch_types=[
            pltpu.VMEM((GITER, GCH), jnp.int32),
            pltpu.VMEM((GCH, H), jnp.float32),
            pltpu.SemaphoreType.DMA,
        ],
    )
    def gk(emb_hbm, idx_hbm, out_hbm, idx_v, rows_v, sem):
        wid = lax.axis_index("s") * SC_NC + lax.axis_index("c")
        pltpu.sync_copy(idx_hbm.at[pl.ds(wid * GITER, GITER)], idx_v)

        def body(j, carry):
            pltpu.async_copy(emb_hbm.at[idx_v.at[j]], rows_v, sem).wait()
            pltpu.sync_copy(rows_v, out_hbm.at[pl.ds(wid * RPW + j * GCH, GCH)])
            return carry

        lax.fori_loop(0, GITER, body, 0)

    return gk(emb, idx2d)


def _mm(a, b):
    return lax.dot_general(a, b, (((1,), (1,)), ((), ())),
                           preferred_element_type=jnp.float32)


def _gates(iou, c_pre):
    i = iou[:, 0:H]
    o = iou[:, H:2 * H]
    u = iou[:, 2 * H:3 * H]
    c = jax.nn.sigmoid(i) * jnp.tanh(u) + c_pre
    h = jax.nn.sigmoid(o) * jnp.tanh(c)
    return h, c


def _leaf_body(x_ref, wiou_ref, biou_ref, ufw_ref, ufb_ref,
               hsum_ref, ht_ref, fc_ref):
    j = pl.program_id(1)

    @pl.when(j == 0)
    def _():
        hsum_ref[...] = jnp.zeros(hsum_ref.shape, jnp.float32)
        ht_ref[...] = jnp.zeros(ht_ref.shape, jnp.float32)
        fc_ref[...] = jnp.zeros(fc_ref.shape, jnp.float32)

    x = x_ref[0]
    iou = _mm(x, wiou_ref[...]) + biou_ref[...]
    h, c = _gates(iou, 0.0)
    rows = GS + CHUNK * j + lax.broadcasted_iota(jnp.int32, (CHUNK, 1), 0)
    mask = (rows >= LEAF0) & (rows < N)
    h = jnp.where(mask, h, 0.0)
    c = jnp.where(mask, c, 0.0)
    f = jax.nn.sigmoid(_mm(h, ufw_ref[...]) + ufb_ref[...])
    hsum_ref[0] += jnp.reshape(h, (CHUNK // 8, 8, H)).sum(axis=0)
    ht_ref[0, pl.ds(33 + K * j, K), :] = jnp.reshape(h, (K, K, H)).sum(axis=1)
    fc_ref[0, pl.ds(33 + K * j, K), :] = jnp.reshape(f * c, (K, K, H)).sum(axis=1)


def _finish_body(ht_ref, fc_ref, hsum_ref, uiou_ref, biou_ref, ufw_ref,
                 ufb_ref, cw_ref, cb_ref, out_ref):
    reps = []
    for t in range(2):
        ht = ht_ref[t]
        fcv = fc_ref[t]
        iou3 = _mm(ht[GS:GS + 512], uiou_ref[...]) + biou_ref[...]
        h3, c3 = _gates(iou3, fcv[GS:GS + 512])
        r3 = GS + lax.broadcasted_iota(jnp.int32, (512, 1), 0)
        m3 = r3 < LEAF0
        h3 = jnp.where(m3, h3, 0.0)
        c3 = jnp.where(m3, c3, 0.0)
        f3 = jax.nn.sigmoid(_mm(h3, ufw_ref[...]) + ufb_ref[...])
        add_h = jnp.reshape(h3, (16, K, H)).sum(axis=1)
        add_fc = jnp.reshape(f3 * c3, (16, K, H)).sum(axis=1)
        pad = jnp.zeros((1024 - 16, H), jnp.float32)
        htild2 = ht[33:1057] + jnp.concatenate([add_h, pad], axis=0)
        fc2 = fcv[33:1057] + jnp.concatenate([add_fc, pad], axis=0)
        iou2 = _mm(htild2, uiou_ref[...]) + biou_ref[...]
        h2, c2 = _gates(iou2, fc2)
        f2 = jax.nn.sigmoid(_mm(h2, ufw_ref[...]) + ufb_ref[...])
        htild1 = jnp.reshape(h2, (K, K, H)).sum(axis=1)
        fc1 = jnp.reshape(f2 * c2, (K, K, H)).sum(axis=1)
        iou1 = _mm(htild1, uiou_ref[...]) + biou_ref[...]
        h1, c1 = _gates(iou1, fc1)
        f1 = jax.nn.sigmoid(_mm(h1, ufw_ref[...]) + ufb_ref[...])
        htild0 = jnp.sum(h1, axis=0, keepdims=True)
        fc0 = jnp.sum(f1 * c1, axis=0, keepdims=True)
        iou0 = _mm(htild0, uiou_ref[...]) + biou_ref[...]
        h0, _ = _gates(iou0, fc0)
        tot = (jnp.sum(hsum_ref[t], axis=0, keepdims=True)
               + jnp.sum(h3, axis=0, keepdims=True)
               + jnp.sum(h2, axis=0, keepdims=True)
               + jnp.sum(h1, axis=0, keepdims=True)
               + h0)
        reps.append(tot * (1.0 / N))
    ad = jnp.abs(reps[0] - reps[1])
    out_ref[...] = _mm(ad, cw_ref[...]) + cb_ref[...]


def _leaf_call(x_all, W_iou, b_iou, U_f_w, U_f_b2):
    return pl.pallas_call(
        _leaf_body,
        grid=(2, NCHUNK),
        in_specs=[
            pl.BlockSpec((1, CHUNK, H), lambda t, j: (t, j, 0)),
            pl.BlockSpec((3 * H, H), lambda t, j: (0, 0)),
            pl.BlockSpec((1, 3 * H), lambda t, j: (0, 0)),
            pl.BlockSpec((H, H), lambda t, j: (0, 0)),
            pl.BlockSpec((1, H), lambda t, j: (0, 0)),
        ],
        out_specs=[
            pl.BlockSpec((1, 8, H), lambda t, j: (t, 0, 0)),
            pl.BlockSpec((1, PA, H), lambda t, j: (t, 0, 0)),
            pl.BlockSpec((1, PA, H), lambda t, j: (t, 0, 0)),
        ],
        out_shape=[
            jax.ShapeDtypeStruct((2, 8, H), jnp.float32),
            jax.ShapeDtypeStruct((2, PA, H), jnp.float32),
            jax.ShapeDtypeStruct((2, PA, H), jnp.float32),
        ],
    )(x_all, W_iou, b_iou, U_f_w, U_f_b2)


def _finish_call(ht, fc, hsum, U_iou, b_iou, U_f_w, U_f_b2, cw_pad, cb_pad):
    return pl.pallas_call(
        _finish_body,
        out_shape=jax.ShapeDtypeStruct((1, H), jnp.float32),
    )(ht, fc, hsum, U_iou, b_iou, U_f_w, U_f_b2, cw_pad, cb_pad)


def kernel(types1, types2, emb, W_iou, U_iou, b_iou, U_f_w, U_f_b,
           classify_w, classify_b):
    t1 = lax.slice(types1, (GS,), (N,)).astype(jnp.int32)
    t2 = lax.slice(types2, (GS,), (N,)).astype(jnp.int32)
    zpad = jnp.zeros((GL - (N - GS),), jnp.int32)
    idx2d = jnp.concatenate([t1, zpad, t2, zpad]).reshape(2 * GL // GCH, GCH)

    x_all = _gather_rows(emb, idx2d).reshape(2, GL, H)

    U_f_b2 = U_f_b.reshape(1, H)
    hsum, ht, fc = _leaf_call(x_all, W_iou, b_iou, U_f_w, U_f_b2)

    cw_pad = jnp.pad(classify_w, ((0, H - 2), (0, 0)))
    cb_pad = jnp.pad(classify_b.reshape(1, 2), ((0, 0), (0, H - 2)))
    out = _finish_call(ht, fc, hsum, U_iou, b_iou, U_f_w, U_f_b2,
                       cw_pad, cb_pad)
    return out[:, :2]


# speedup vs baseline: 17.3184x; 1.1807x over previous
import functools

import jax
import jax.numpy as jnp
from jax import lax
from jax.experimental import pallas as pl
from jax.experimental.pallas import tpu as pltpu
from jax.experimental.pallas import tpu_sc as plsc

N = 50000
H = 128
K = 32
GS = 1057
NCHUNK = 48
CHUNK = 1024
GL = NCHUNK * CHUNK
PA = 1600
LEAF0 = 1563

SC_NC = 2
SC_NS = 16
NW = SC_NC * SC_NS
RPW = GL // NW
GCH = 128
GITER = RPW // GCH
GITER_PAD = 16


def _gather_rows(emb, idx2d):
    mesh = plsc.VectorSubcoreMesh(core_axis_name="c", subcore_axis_name="s")

    @functools.partial(
        pl.kernel,
        mesh=mesh,
        out_type=jax.ShapeDtypeStruct((GL, H), jnp.float32),
        scratch_types=[
            pltpu.VMEM((GITER_PAD, GCH), jnp.int32),
            pltpu.VMEM((2, GCH, H), jnp.float32),
            pltpu.SemaphoreType.DMA,
            pltpu.SemaphoreType.DMA,
        ],
    )
    def gk(emb_hbm, idx_hbm, out_hbm, idx_v, rows_v, sem_a, sem_b):
        wid = lax.axis_index("s") * SC_NC + lax.axis_index("c")
        base = wid * RPW
        pltpu.sync_copy(idx_hbm.at[pl.ds(wid * GITER_PAD, GITER_PAD)], idx_v)
        pltpu.make_async_copy(
            emb_hbm.at[idx_v.at[0]], rows_v.at[0], sem_a).start()

        def body(p, carry):
            j0 = 2 * p
            j1 = j0 + 1
            pltpu.make_async_copy(
                emb_hbm.at[idx_v.at[j1]], rows_v.at[1], sem_b).start()
            pltpu.make_async_copy(
                emb_hbm.at[idx_v.at[j0]], rows_v.at[0], sem_a).wait()
            pltpu.sync_copy(rows_v.at[0],
                            out_hbm.at[pl.ds(base + j0 * GCH, GCH)])
            @pl.when(p + 1 < GITER // 2)
            def _():
                pltpu.make_async_copy(
                    emb_hbm.at[idx_v.at[j0 + 2]], rows_v.at[0], sem_a).start()
            pltpu.make_async_copy(
                emb_hbm.at[idx_v.at[j1]], rows_v.at[1], sem_b).wait()
            pltpu.sync_copy(rows_v.at[1],
                            out_hbm.at[pl.ds(base + j1 * GCH, GCH)])
            return carry

        lax.fori_loop(0, GITER // 2, body, 0)

    return gk(emb, idx2d)


def _mm(a, b):
    return lax.dot_general(a, b, (((1,), (1,)), ((), ())),
                           preferred_element_type=jnp.float32)


def _gates(iou, c_pre):
    i = iou[:, 0:H]
    o = iou[:, H:2 * H]
    u = iou[:, 2 * H:3 * H]
    c = jax.nn.sigmoid(i) * jnp.tanh(u) + c_pre
    h = jax.nn.sigmoid(o) * jnp.tanh(c)
    return h, c


def _leaf_body(x_ref, wiou_ref, biou_ref, ufw_ref, ufb_ref,
               hsum_ref, ht_ref, fc_ref):
    j = pl.program_id(0)

    @pl.when(j == 0)
    def _():
        hsum_ref[...] = jnp.zeros(hsum_ref.shape, jnp.float32)
        ht_ref[...] = jnp.zeros(ht_ref.shape, jnp.float32)
        fc_ref[...] = jnp.zeros(fc_ref.shape, jnp.float32)

    x = x_ref[...]
    iou = _mm(x, wiou_ref[...]) + biou_ref[...]
    h, c = _gates(iou, 0.0)
    rows = GS + CHUNK * j + lax.broadcasted_iota(jnp.int32, (CHUNK, 1), 0)
    mask = (rows >= LEAF0) & (rows < N)
    h = jnp.where(mask, h, 0.0)
    c = jnp.where(mask, c, 0.0)
    f = jax.nn.sigmoid(_mm(h, ufw_ref[...]) + ufb_ref[...])
    hsum_ref[...] += jnp.reshape(h, (CHUNK // 8, 8, H)).sum(axis=0)
    ht_ref[pl.ds(33 + K * j, K), :] = jnp.reshape(h, (K, K, H)).sum(axis=1)
    fc_ref[pl.ds(33 + K * j, K), :] = jnp.reshape(f * c, (K, K, H)).sum(axis=1)


def _finish_body(ht1_ref, fc1_ref, hs1_ref, ht2_ref, fc2_ref, hs2_ref,
                 uiou_ref, biou_ref, ufw_ref, ufb_ref, cw_ref, cb_ref,
                 out_ref):
    reps = []
    for ht_ref, fcv_ref, hsum_ref in ((ht1_ref, fc1_ref, hs1_ref),
                                      (ht2_ref, fc2_ref, hs2_ref)):
        ht = ht_ref[...]
        fcv = fcv_ref[...]
        iou3 = _mm(ht[GS:GS + 512], uiou_ref[...]) + biou_ref[...]
        h3, c3 = _gates(iou3, fcv[GS:GS + 512])
        r3 = GS + lax.broadcasted_iota(jnp.int32, (512, 1), 0)
        m3 = r3 < LEAF0
        h3 = jnp.where(m3, h3, 0.0)
        c3 = jnp.where(m3, c3, 0.0)
        f3 = jax.nn.sigmoid(_mm(h3, ufw_ref[...]) + ufb_ref[...])
        add_h = jnp.reshape(h3, (16, K, H)).sum(axis=1)
        add_fc = jnp.reshape(f3 * c3, (16, K, H)).sum(axis=1)
        pad = jnp.zeros((1024 - 16, H), jnp.float32)
        htild2 = ht[33:1057] + jnp.concatenate([add_h, pad], axis=0)
        fc2 = fcv[33:1057] + jnp.concatenate([add_fc, pad], axis=0)
        iou2 = _mm(htild2, uiou_ref[...]) + biou_ref[...]
        h2, c2 = _gates(iou2, fc2)
        f2 = jax.nn.sigmoid(_mm(h2, ufw_ref[...]) + ufb_ref[...])
        htild1 = jnp.reshape(h2, (K, K, H)).sum(axis=1)
        fc1 = jnp.reshape(f2 * c2, (K, K, H)).sum(axis=1)
        iou1 = _mm(htild1, uiou_ref[...]) + biou_ref[...]
        h1, c1 = _gates(iou1, fc1)
        f1 = jax.nn.sigmoid(_mm(h1, ufw_ref[...]) + ufb_ref[...])
        htild0 = jnp.sum(h1, axis=0, keepdims=True)
        fc0 = jnp.sum(f1 * c1, axis=0, keepdims=True)
        iou0 = _mm(htild0, uiou_ref[...]) + biou_ref[...]
        h0, _ = _gates(iou0, fc0)
        tot = (jnp.sum(hsum_ref[...], axis=0, keepdims=True)
               + jnp.sum(h3, axis=0, keepdims=True)
               + jnp.sum(h2, axis=0, keepdims=True)
               + jnp.sum(h1, axis=0, keepdims=True)
               + h0)
        reps.append(tot * (1.0 / N))
    ad = jnp.abs(reps[0] - reps[1])
    out_ref[...] = _mm(ad, cw_ref[...]) + cb_ref[...]


def _leaf_call(x, W_iou, b_iou, U_f_w, U_f_b2):
    return pl.pallas_call(
        _leaf_body,
        grid=(NCHUNK,),
        in_specs=[
            pl.BlockSpec((CHUNK, H), lambda j: (j, 0)),
            pl.BlockSpec((3 * H, H), lambda j: (0, 0)),
            pl.BlockSpec((1, 3 * H), lambda j: (0, 0)),
            pl.BlockSpec((H, H), lambda j: (0, 0)),
            pl.BlockSpec((1, H), lambda j: (0, 0)),
        ],
        out_specs=[
            pl.BlockSpec((8, H), lambda j: (0, 0)),
            pl.BlockSpec((PA, H), lambda j: (0, 0)),
            pl.BlockSpec((PA, H), lambda j: (0, 0)),
        ],
        out_shape=[
            jax.ShapeDtypeStruct((8, H), jnp.float32),
            jax.ShapeDtypeStruct((PA, H), jnp.float32),
            jax.ShapeDtypeStruct((PA, H), jnp.float32),
        ],
    )(x, W_iou, b_iou, U_f_w, U_f_b2)


def _finish_call(l1, l2, U_iou, b_iou, U_f_w, U_f_b2, cw_pad, cb_pad):
    hs1, ht1, fc1 = l1
    hs2, ht2, fc2 = l2
    return pl.pallas_call(
        _finish_body,
        out_shape=jax.ShapeDtypeStruct((1, H), jnp.float32),
    )(ht1, fc1, hs1, ht2, fc2, hs2, U_iou, b_iou, U_f_w, U_f_b2,
      cw_pad, cb_pad)


def kernel(types1, types2, emb, W_iou, U_iou, b_iou, U_f_w, U_f_b,
           classify_w, classify_b):
    zpad = jnp.zeros((GL - (N - GS),), jnp.int32)

    def _mk_idx(types):
        t = lax.slice(types, (GS,), (N,)).astype(jnp.int32)
        i3 = jnp.concatenate([t, zpad]).reshape(NW, GITER, GCH)
        i3 = jnp.pad(i3, ((0, 0), (0, GITER_PAD - GITER), (0, 0)))
        return i3.reshape(NW * GITER_PAD, GCH)

    idx1 = _mk_idx(types1)
    idx2 = _mk_idx(types2)

    U_f_b2 = U_f_b.reshape(1, H)
    x1 = _gather_rows(emb, idx1)
    x2 = _gather_rows(emb, idx2)
    l1 = _leaf_call(x1, W_iou, b_iou, U_f_w, U_f_b2)
    l2 = _leaf_call(x2, W_iou, b_iou, U_f_w, U_f_b2)

    cw_pad = jnp.pad(classify_w, ((0, H - 2), (0, 0)))
    cb_pad = jnp.pad(classify_b.reshape(1, 2), ((0, 0), (0, H - 2)))
    out = _finish_call(l1, l2, U_iou, b_iou, U_f_w, U_f_b2, cw_pad, cb_pad)
    return out[:, :2]


# speedup vs baseline: 17.3882x; 1.0040x over previous
import functools

import jax
import jax.numpy as jnp
from jax import lax
from jax.experimental import pallas as pl
from jax.experimental.pallas import tpu as pltpu
from jax.experimental.pallas import tpu_sc as plsc

N = 50000
H = 128
K = 32
GS = 1057
NCHUNK = 48
CHUNK = 1024
GL = NCHUNK * CHUNK
PA = 1600
LEAF0 = 1563

SC_NC = 2
SC_NS = 16
NW = SC_NC * SC_NS
RPW = GL // NW
GCH = 128
GITER = RPW // GCH
GITER_PAD = 16


def _gather_rows(emb, idx2d):
    mesh = plsc.VectorSubcoreMesh(core_axis_name="c", subcore_axis_name="s")

    @functools.partial(
        pl.kernel,
        mesh=mesh,
        out_type=jax.ShapeDtypeStruct((GL, H), jnp.float32),
        scratch_types=[
            pltpu.VMEM((GITER_PAD, GCH), jnp.int32),
            pltpu.VMEM((2, GCH, H), jnp.float32),
            pltpu.SemaphoreType.DMA,
            pltpu.SemaphoreType.DMA,
        ],
    )
    def gk(emb_hbm, idx_hbm, out_hbm, idx_v, rows_v, sem_a, sem_b):
        wid = lax.axis_index("s") * SC_NC + lax.axis_index("c")
        base = wid * RPW
        pltpu.sync_copy(idx_hbm.at[pl.ds(wid * GITER_PAD, GITER_PAD)], idx_v)
        pltpu.make_async_copy(
            emb_hbm.at[idx_v.at[0]], rows_v.at[0], sem_a).start()

        def body(p, carry):
            j0 = 2 * p
            j1 = j0 + 1
            pltpu.make_async_copy(
                emb_hbm.at[idx_v.at[j1]], rows_v.at[1], sem_b).start()
            pltpu.make_async_copy(
                emb_hbm.at[idx_v.at[j0]], rows_v.at[0], sem_a).wait()
            pltpu.sync_copy(rows_v.at[0],
                            out_hbm.at[pl.ds(base + j0 * GCH, GCH)])
            @pl.when(p + 1 < GITER // 2)
            def _():
                pltpu.make_async_copy(
                    emb_hbm.at[idx_v.at[j0 + 2]], rows_v.at[0], sem_a).start()
            pltpu.make_async_copy(
                emb_hbm.at[idx_v.at[j1]], rows_v.at[1], sem_b).wait()
            pltpu.sync_copy(rows_v.at[1],
                            out_hbm.at[pl.ds(base + j1 * GCH, GCH)])
            return carry

        lax.fori_loop(0, GITER // 2, body, 0)

    return gk(emb, idx2d)


def _mm(a, b):
    return lax.dot_general(a, b, (((1,), (1,)), ((), ())),
                           preferred_element_type=jnp.float32)


def _smm(s, a):
    return lax.dot_general(s, a, (((1,), (0,)), ((), ())),
                           preferred_element_type=jnp.float32)


def _sig(x):
    return jnp.tanh(x * 0.5) * 0.5 + 0.5


def _gates(iou, c_pre):
    i = iou[:, 0:H]
    o = iou[:, H:2 * H]
    u = iou[:, 2 * H:3 * H]
    c = _sig(i) * jnp.tanh(u) + c_pre
    h = _sig(o) * jnp.tanh(c)
    return h, c


def _leaf_body(x_ref, wiou_ref, biou_ref, ufw_ref, ufb_ref, seg_ref,
               hsum_ref, ht_ref, fc_ref):
    j = pl.program_id(0)

    @pl.when(j == 0)
    def _():
        hsum_ref[...] = jnp.zeros(hsum_ref.shape, jnp.float32)
        ht_ref[...] = jnp.zeros(ht_ref.shape, jnp.float32)
        fc_ref[...] = jnp.zeros(fc_ref.shape, jnp.float32)

    x = x_ref[...]
    iou = _mm(x, wiou_ref[...]) + biou_ref[...]
    h, c = _gates(iou, 0.0)
    rows = GS + CHUNK * j + lax.broadcasted_iota(jnp.int32, (CHUNK, 1), 0)
    mask = (rows >= LEAF0) & (rows < N)
    h = jnp.where(mask, h, 0.0)
    c = jnp.where(mask, c, 0.0)
    f = _sig(_mm(h, ufw_ref[...]) + ufb_ref[...])
    seg = seg_ref[...]
    rh = _smm(seg, h)
    rf = _smm(seg, f * c)
    hsum_ref[...] += rh[32:40]
    ht_ref[pl.ds(33 + K * j, K), :] = rh[0:32]
    fc_ref[pl.ds(33 + K * j, K), :] = rf[0:32]


def _finish_body(ht1_ref, fc1_ref, hs1_ref, ht2_ref, fc2_ref, hs2_ref,
                 uiou_ref, biou_ref, ufw_ref, ufb_ref, cw_ref, cb_ref,
                 out_ref):
    reps = []
    for ht_ref, fcv_ref, hsum_ref in ((ht1_ref, fc1_ref, hs1_ref),
                                      (ht2_ref, fc2_ref, hs2_ref)):
        ht = ht_ref[...]
        fcv = fcv_ref[...]
        iou3 = _mm(ht[GS:GS + 512], uiou_ref[...]) + biou_ref[...]
        h3, c3 = _gates(iou3, fcv[GS:GS + 512])
        r3 = GS + lax.broadcasted_iota(jnp.int32, (512, 1), 0)
        m3 = r3 < LEAF0
        h3 = jnp.where(m3, h3, 0.0)
        c3 = jnp.where(m3, c3, 0.0)
        f3 = _sig(_mm(h3, ufw_ref[...]) + ufb_ref[...])
        add_h = jnp.reshape(h3, (16, K, H)).sum(axis=1)
        add_fc = jnp.reshape(f3 * c3, (16, K, H)).sum(axis=1)
        pad = jnp.zeros((1024 - 16, H), jnp.float32)
        htild2 = ht[33:1057] + jnp.concatenate([add_h, pad], axis=0)
        fc2 = fcv[33:1057] + jnp.concatenate([add_fc, pad], axis=0)
        iou2 = _mm(htild2, uiou_ref[...]) + biou_ref[...]
        h2, c2 = _gates(iou2, fc2)
        f2 = _sig(_mm(h2, ufw_ref[...]) + ufb_ref[...])
        htild1 = jnp.reshape(h2, (K, K, H)).sum(axis=1)
        fc1 = jnp.reshape(f2 * c2, (K, K, H)).sum(axis=1)
        iou1 = _mm(htild1, uiou_ref[...]) + biou_ref[...]
        h1, c1 = _gates(iou1, fc1)
        f1 = _sig(_mm(h1, ufw_ref[...]) + ufb_ref[...])
        htild0 = jnp.sum(h1, axis=0, keepdims=True)
        fc0 = jnp.sum(f1 * c1, axis=0, keepdims=True)
        iou0 = _mm(htild0, uiou_ref[...]) + biou_ref[...]
        h0, _ = _gates(iou0, fc0)
        tot = (jnp.sum(hsum_ref[...], axis=0, keepdims=True)
               + jnp.sum(h3, axis=0, keepdims=True)
               + jnp.sum(h2, axis=0, keepdims=True)
               + jnp.sum(h1, axis=0, keepdims=True)
               + h0)
        reps.append(tot * (1.0 / N))
    ad = jnp.abs(reps[0] - reps[1])
    out_ref[...] = _mm(ad, cw_ref[...]) + cb_ref[...]


def _leaf_call(x, W_iou, b_iou, U_f_w, U_f_b2, seg):
    return pl.pallas_call(
        _leaf_body,
        grid=(NCHUNK,),
        in_specs=[
            pl.BlockSpec((CHUNK, H), lambda j: (j, 0)),
            pl.BlockSpec((3 * H, H), lambda j: (0, 0)),
            pl.BlockSpec((1, 3 * H), lambda j: (0, 0)),
            pl.BlockSpec((H, H), lambda j: (0, 0)),
            pl.BlockSpec((1, H), lambda j: (0, 0)),
            pl.BlockSpec((40, CHUNK), lambda j: (0, 0)),
        ],
        out_specs=[
            pl.BlockSpec((8, H), lambda j: (0, 0)),
            pl.BlockSpec((PA, H), lambda j: (0, 0)),
            pl.BlockSpec((PA, H), lambda j: (0, 0)),
        ],
        out_shape=[
            jax.ShapeDtypeStruct((8, H), jnp.float32),
            jax.ShapeDtypeStruct((PA, H), jnp.float32),
            jax.ShapeDtypeStruct((PA, H), jnp.float32),
        ],
    )(x, W_iou, b_iou, U_f_w, U_f_b2, seg)


def _finish_call(l1, l2, U_iou, b_iou, U_f_w, U_f_b2, cw_pad, cb_pad):
    hs1, ht1, fc1 = l1
    hs2, ht2, fc2 = l2
    return pl.pallas_call(
        _finish_body,
        out_shape=jax.ShapeDtypeStruct((1, H), jnp.float32),
    )(ht1, fc1, hs1, ht2, fc2, hs2, U_iou, b_iou, U_f_w, U_f_b2,
      cw_pad, cb_pad)


def kernel(types1, types2, emb, W_iou, U_iou, b_iou, U_f_w, U_f_b,
           classify_w, classify_b):
    zpad = jnp.zeros((GL - (N - GS),), jnp.int32)

    def _mk_idx(types):
        t = lax.slice(types, (GS,), (N,)).astype(jnp.int32)
        i3 = jnp.concatenate([t, zpad]).reshape(NW, GITER, GCH)
        i3 = jnp.pad(i3, ((0, 0), (0, GITER_PAD - GITER), (0, 0)))
        return i3.reshape(NW * GITER_PAD, GCH)

    idx1 = _mk_idx(types1)
    idx2 = _mk_idx(types2)

    U_f_b2 = U_f_b.reshape(1, H)
    blk = lax.broadcasted_iota(jnp.int32, (40, CHUNK), 0)
    pos = lax.broadcasted_iota(jnp.int32, (40, CHUNK), 1) // K
    seg = ((blk == pos) | (blk == 32)).astype(jnp.float32)
    x1 = _gather_rows(emb, idx1)
    x2 = _gather_rows(emb, idx2)
    l1 = _leaf_call(x1, W_iou, b_iou, U_f_w, U_f_b2, seg)
    l2 = _leaf_call(x2, W_iou, b_iou, U_f_w, U_f_b2, seg)

    cw_pad = jnp.pad(classify_w, ((0, H - 2), (0, 0)))
    cb_pad = jnp.pad(classify_b.reshape(1, 2), ((0, 0), (0, H - 2)))
    out = _finish_call(l1, l2, U_iou, b_iou, U_f_w, U_f_b2, cw_pad, cb_pad)
    return out[:, :2]


# speedup vs baseline: 17.9461x; 1.0321x over previous
import functools

import numpy as np
import jax
import jax.numpy as jnp
from jax import lax
from jax.experimental import pallas as pl
from jax.experimental.pallas import tpu as pltpu
from jax.experimental.pallas import tpu_sc as plsc

N = 50000
H = 128
K = 32
GS = 1057
CHUNK = 1024
NCH_H = 24
HL = NCH_H * CHUNK
PR = NCH_H * K
LEAF0 = 1563

SC_NC = 2
SC_NS = 16
NW = SC_NC * SC_NS
RPW = HL // NW
GCH = 128
GITER = RPW // GCH
GITER_PAD = 8

_SEG_NP = np.zeros((40, CHUNK), np.float32)
for _p in range(K):
    _SEG_NP[_p, _p * K:(_p + 1) * K] = 0.5
_SEG_NP[K, :] = 0.5


def _gather_rows(emb, idx2d):
    mesh = plsc.VectorSubcoreMesh(core_axis_name="c", subcore_axis_name="s")

    @functools.partial(
        pl.kernel,
        mesh=mesh,
        out_type=jax.ShapeDtypeStruct((HL, H), jnp.float32),
        scratch_types=[
            pltpu.VMEM((GITER_PAD, GCH), jnp.int32),
            pltpu.VMEM((2, GCH, H), jnp.float32),
            pltpu.SemaphoreType.DMA,
            pltpu.SemaphoreType.DMA,
        ],
    )
    def gk(emb_hbm, idx_hbm, out_hbm, idx_v, rows_v, sem_a, sem_b):
        wid = lax.axis_index("s") * SC_NC + lax.axis_index("c")
        base = wid * RPW
        pltpu.sync_copy(idx_hbm.at[pl.ds(wid * GITER_PAD, GITER_PAD)], idx_v)
        pltpu.make_async_copy(
            emb_hbm.at[idx_v.at[0]], rows_v.at[0], sem_a).start()

        def body(p, carry):
            j0 = 2 * p
            j1 = j0 + 1
            pltpu.make_async_copy(
                emb_hbm.at[idx_v.at[j1]], rows_v.at[1], sem_b).start()
            pltpu.make_async_copy(
                emb_hbm.at[idx_v.at[j0]], rows_v.at[0], sem_a).wait()
            pltpu.sync_copy(rows_v.at[0],
                            out_hbm.at[pl.ds(base + j0 * GCH, GCH)])
            @pl.when(p + 1 < GITER // 2)
            def _():
                pltpu.make_async_copy(
                    emb_hbm.at[idx_v.at[j0 + 2]], rows_v.at[0], sem_a).start()
            pltpu.make_async_copy(
                emb_hbm.at[idx_v.at[j1]], rows_v.at[1], sem_b).wait()
            pltpu.sync_copy(rows_v.at[1],
                            out_hbm.at[pl.ds(base + j1 * GCH, GCH)])
            return carry

        lax.fori_loop(0, GITER // 2, body, 0)

    return gk(emb, idx2d)


def _mm(a, b):
    return lax.dot_general(a, b, (((1,), (1,)), ((), ())),
                           preferred_element_type=jnp.float32)


def _smm(s, a):
    return lax.dot_general(s, a, (((1,), (0,)), ((), ())),
                           preferred_element_type=jnp.float32)


def _sig(x):
    return jnp.tanh(x * 0.5) * 0.5 + 0.5


def _gates(iou, c_pre):
    i = iou[:, 0:H]
    o = iou[:, H:2 * H]
    u = iou[:, 2 * H:3 * H]
    c = _sig(i) * jnp.tanh(u) + c_pre
    h = _sig(o) * jnp.tanh(c)
    return h, c


def _leaf_body(base_row, x_ref, wiou_ref, biou_ref, ufw_ref, ufb_ref,
               seg_ref, hsum_ref, ht_ref, fc_ref):
    j = pl.program_id(0)

    @pl.when(j == 0)
    def _():
        hsum_ref[...] = jnp.zeros(hsum_ref.shape, jnp.float32)

    x = x_ref[...]
    iou = _mm(x, wiou_ref[...]) + biou_ref[...]
    t = jnp.tanh(iou)
    t_i = t[:, 0:H]
    t_o = t[:, H:2 * H]
    t_u = t[:, 2 * H:3 * H]
    c = (t_i + 1.0) * (t_u * 0.5)
    h2 = (t_o + 1.0) * jnp.tanh(c)
    rows = base_row + CHUNK * j + lax.broadcasted_iota(jnp.int32, (CHUNK, 1), 0)
    mask = (rows >= LEAF0) & (rows < N)
    h2 = jnp.where(mask, h2, 0.0)
    c = jnp.where(mask, c, 0.0)
    t_f = jnp.tanh(_mm(h2, ufw_ref[...]) + ufb_ref[...])
    fc2 = (t_f + 1.0) * c
    seg = seg_ref[...]
    rh = _smm(seg, h2)
    rf = _smm(seg, fc2)
    hsum_ref[...] += rh[32:40]
    ht_ref[pl.ds(K * j, K), :] = rh[0:32]
    fc_ref[pl.ds(K * j, K), :] = rf[0:32]


def _leaf_call(half, x, wiou_s, biou_s, ufw_s, ufb_s, seg):
    base_row = GS + HL * half
    return pl.pallas_call(
        functools.partial(_leaf_body, base_row),
        grid=(NCH_H,),
        in_specs=[
            pl.BlockSpec((CHUNK, H), lambda j: (j, 0)),
            pl.BlockSpec((3 * H, H), lambda j: (0, 0)),
            pl.BlockSpec((1, 3 * H), lambda j: (0, 0)),
            pl.BlockSpec((H, H), lambda j: (0, 0)),
            pl.BlockSpec((1, H), lambda j: (0, 0)),
            pl.BlockSpec((40, CHUNK), lambda j: (0, 0)),
        ],
        out_specs=[
            pl.BlockSpec((8, H), lambda j: (0, 0)),
            pl.BlockSpec((PR, H), lambda j: (0, 0)),
            pl.BlockSpec((PR, H), lambda j: (0, 0)),
        ],
        out_shape=[
            jax.ShapeDtypeStruct((8, H), jnp.float32),
            jax.ShapeDtypeStruct((PR, H), jnp.float32),
            jax.ShapeDtypeStruct((PR, H), jnp.float32),
        ],
    )(x, wiou_s, biou_s, ufw_s, ufb_s, seg)


def _finish_body(hta1, htb1, fca1, fcb1, hsa1, hsb1,
                 hta2, htb2, fca2, fcb2, hsa2, hsb2,
                 uiou_ref, biou_ref, ufw_ref, ufb_ref, cw_ref, cb_ref,
                 out_ref):
    reps = []
    for hta, htb, fca, fcb, hsa, hsb in (
            (hta1, htb1, fca1, fcb1, hsa1, hsb1),
            (hta2, htb2, fca2, fcb2, hsa2, hsb2)):
        iou3 = _mm(htb[256:768], uiou_ref[...]) + biou_ref[...]
        h3, c3 = _gates(iou3, fcb[256:768])
        r3 = GS + lax.broadcasted_iota(jnp.int32, (512, 1), 0)
        m3 = r3 < LEAF0
        h3 = jnp.where(m3, h3, 0.0)
        c3 = jnp.where(m3, c3, 0.0)
        f3 = _sig(_mm(h3, ufw_ref[...]) + ufb_ref[...])
        add_h = jnp.reshape(h3, (16, K, H)).sum(axis=1)
        add_fc = jnp.reshape(f3 * c3, (16, K, H)).sum(axis=1)
        htild2 = jnp.concatenate(
            [hta[0:16] + add_h, hta[16:768], htb[0:256]], axis=0)
        fc2 = jnp.concatenate(
            [fca[0:16] + add_fc, fca[16:768], fcb[0:256]], axis=0)
        iou2 = _mm(htild2, uiou_ref[...]) + biou_ref[...]
        h2, c2 = _gates(iou2, fc2)
        f2 = _sig(_mm(h2, ufw_ref[...]) + ufb_ref[...])
        htild1 = jnp.reshape(h2, (K, K, H)).sum(axis=1)
        fc1 = jnp.reshape(f2 * c2, (K, K, H)).sum(axis=1)
        iou1 = _mm(htild1, uiou_ref[...]) + biou_ref[...]
        h1, c1 = _gates(iou1, fc1)
        f1 = _sig(_mm(h1, ufw_ref[...]) + ufb_ref[...])
        htild0 = jnp.sum(h1, axis=0, keepdims=True)
        fc0 = jnp.sum(f1 * c1, axis=0, keepdims=True)
        iou0 = _mm(htild0, uiou_ref[...]) + biou_ref[...]
        h0, _ = _gates(iou0, fc0)
        tot = (jnp.sum(hsa[...] + hsb[...], axis=0, keepdims=True)
               + jnp.sum(h3, axis=0, keepdims=True)
               + jnp.sum(h2, axis=0, keepdims=True)
               + jnp.sum(h1, axis=0, keepdims=True)
               + h0)
        reps.append(tot * (1.0 / N))
    ad = jnp.abs(reps[0] - reps[1])
    out_ref[...] = _mm(ad, cw_ref[...]) + cb_ref[...]


def _finish_call(l1a, l1b, l2a, l2b, U_iou, b_iou, U_f_w, U_f_b2,
                 cw_pad, cb_pad):
    return pl.pallas_call(
        _finish_body,
        out_shape=jax.ShapeDtypeStruct((1, H), jnp.float32),
    )(l1a[1], l1b[1], l1a[2], l1b[2], l1a[0], l1b[0],
      l2a[1], l2b[1], l2a[2], l2b[2], l2a[0], l2b[0],
      U_iou, b_iou, U_f_w, U_f_b2, cw_pad, cb_pad)


def kernel(types1, types2, emb, W_iou, U_iou, b_iou, U_f_w, U_f_b,
           classify_w, classify_b):
    zpad = jnp.zeros((2 * HL - (N - GS),), jnp.int32)

    def _mk_idx(flat):
        i3 = flat.reshape(NW, GITER, GCH)
        i3 = jnp.pad(i3, ((0, 0), (0, GITER_PAD - GITER), (0, 0)))
        return i3.reshape(NW * GITER_PAD, GCH)

    def _mk_halves(types):
        t = lax.slice(types, (GS,), (N,)).astype(jnp.int32)
        t = jnp.concatenate([t, zpad])
        return _mk_idx(lax.slice(t, (0,), (HL,))), \
            _mk_idx(lax.slice(t, (HL,), (2 * HL,)))

    idx1a, idx1b = _mk_halves(types1)
    idx2a, idx2b = _mk_halves(types2)

    wiou_s = jnp.concatenate([W_iou[0:2 * H] * 0.5, W_iou[2 * H:]], axis=0)
    biou_s = jnp.concatenate([b_iou[:, 0:2 * H] * 0.5, b_iou[:, 2 * H:]],
                             axis=1)
    ufw_s = U_f_w * 0.25
    ufb_s = (U_f_b * 0.5).reshape(1, H)
    seg = jnp.asarray(_SEG_NP)
    U_f_b2 = U_f_b.reshape(1, H)

    x1a = _gather_rows(emb, idx1a)
    x1b = _gather_rows(emb, idx1b)
    x2a = _gather_rows(emb, idx2a)
    x2b = _gather_rows(emb, idx2b)
    l1a = _leaf_call(0, x1a, wiou_s, biou_s, ufw_s, ufb_s, seg)
    l1b = _leaf_call(1, x1b, wiou_s, biou_s, ufw_s, ufb_s, seg)
    l2a = _leaf_call(0, x2a, wiou_s, biou_s, ufw_s, ufb_s, seg)
    l2b = _leaf_call(1, x2b, wiou_s, biou_s, ufw_s, ufb_s, seg)

    cw_pad = jnp.pad(classify_w, ((0, H - 2), (0, 0)))
    cb_pad = jnp.pad(classify_b.reshape(1, 2), ((0, 0), (0, H - 2)))
    out = _finish_call(l1a, l1b, l2a, l2b, U_iou, b_iou, U_f_w, U_f_b2,
                       cw_pad, cb_pad)
    return out[:, :2]
